# Initial kernel scaffold; baseline (speedup 1.0000x reference)
import jax
import jax.numpy as jnp
from jax.experimental import pallas as pl


def kernel(timestamps, dow_table, month_table, dom_table, quarter_table):
    raise NotImplementedError("write your pallas kernel here")


# speedup vs baseline: 2.1793x; 2.1793x over previous
import functools

import jax
import jax.numpy as jnp
from jax import lax
from jax.experimental import pallas as pl
from jax.experimental.pallas import tpu as pltpu
from jax.experimental.pallas import tpu_sc as plsc

_B = 16384
_SUB = 32
_NUM_WORKERS = 32
_CHUNK = _B // _NUM_WORKERS
_IDX_PER_WORKER = 4 * _CHUNK
_IDX_CHUNKS = _IDX_PER_WORKER // 128


def _body(ts_hbm, table_hbm, out_hbm, ts_v, idx_v, dst_v, sem):
    wid = lax.axis_index("s") * 2 + lax.axis_index("c")
    base = wid * _CHUNK
    pltpu.sync_copy(ts_hbm.at[pl.ds(base, _CHUNK)], ts_v)

    lanes = lax.iota(jnp.int32, 16)

    def chunk_step(j, carry):
        for g2 in range(2):
            g = 2 * j + g2
            ts = ts_v[pl.ds(g * 16, 16)]
            dn = ts // 86400
            dow = dn % 7
            doy = dn % 365
            month = (doy // 30) % 12
            dom = doy % 31
            quarter = month // 3
            rows = jnp.full((16,), 0, jnp.int32) + j
            colbase = 64 * g2 + 4 * lanes
            plsc.store_scatter(idx_v, [rows, colbase], dow)
            plsc.store_scatter(idx_v, [rows, colbase + 1], month + 7)
            plsc.store_scatter(idx_v, [rows, colbase + 2], dom + 19)
            plsc.store_scatter(idx_v, [rows, colbase + 3], quarter + 50)
        pltpu.async_copy(
            table_hbm.at[idx_v.at[j]],
            dst_v.at[pl.ds(j * 128, 128)],
            sem,
        )
        return carry

    lax.fori_loop(0, _IDX_CHUNKS, chunk_step, 0)

    for _ in range(_IDX_CHUNKS):
        pltpu.make_async_copy(
            table_hbm.at[idx_v.at[0]],
            dst_v.at[pl.ds(0, 128)],
            sem,
        ).wait()

    pltpu.sync_copy(dst_v, out_hbm.at[pl.ds(wid * _IDX_PER_WORKER, _IDX_PER_WORKER)])


@functools.partial(jax.jit)
def _sc_lookup(ts, table):
    mesh = plsc.VectorSubcoreMesh(core_axis_name="c", subcore_axis_name="s")
    k = functools.partial(
        pl.kernel,
        mesh=mesh,
        out_type=jax.ShapeDtypeStruct((4 * _B, _SUB), jnp.float32),
        scratch_types=[
            pltpu.VMEM((_CHUNK,), jnp.int32),
            pltpu.VMEM((_IDX_CHUNKS, 128), jnp.int32),
            pltpu.VMEM((_IDX_PER_WORKER, _SUB), jnp.float32),
            pltpu.SemaphoreType.DMA,
        ],
        compiler_params=pltpu.CompilerParams(
            use_tc_tiling_on_sc=False, needs_layout_passes=False
        ),
    )(_body)
    return k(ts, table)


def kernel(timestamps, dow_table, month_table, dom_table, quarter_table):
    table = jnp.concatenate(
        [dow_table, month_table, dom_table, quarter_table], axis=0
    )
    ts = timestamps.astype(jnp.int32)
    out = _sc_lookup(ts, table)
    return out.reshape(_B, 4 * _SUB)


# speedup vs baseline: 2.2116x; 1.0148x over previous
import functools

import jax
import jax.numpy as jnp
from jax import lax
from jax.experimental import pallas as pl
from jax.experimental.pallas import tpu as pltpu
from jax.experimental.pallas import tpu_sc as plsc

_B = 16384
_SUB = 32
_NUM_WORKERS = 32
_CHUNK = _B // _NUM_WORKERS
_IDX_PER_WORKER = 4 * _CHUNK
_IDX_CHUNKS = _IDX_PER_WORKER // 128


def _body(ts_hbm, table_hbm, out_hbm, ts_v, idx_v, dst_v, sem):
    wid = lax.axis_index("s") * 2 + lax.axis_index("c")
    base = wid * _CHUNK
    pltpu.sync_copy(ts_hbm.at[pl.ds(base, _CHUNK)], ts_v)

    lanes = lax.iota(jnp.int32, 16)

    def chunk_step(j, carry):
        for g2 in range(2):
            g = 2 * j + g2
            ts = ts_v[pl.ds(g * 16, 16)]
            dn0 = (ts.astype(jnp.float32) * jnp.float32(1.0 / 86400.0)).astype(
                jnp.int32
            )
            r = ts - dn0 * 86400
            dn = dn0 + jnp.where(r >= 86400, 1, 0) - jnp.where(r < 0, 1, 0)
            dow = dn - ((dn * 18725) >> 17) * 7
            doy = dn - ((dn * 22983) >> 23) * 365
            q30 = (doy * 1093) >> 15
            month = q30 - jnp.where(q30 >= 12, 12, 0)
            dom = doy - ((doy * 4229) >> 17) * 31
            quarter = (month * 11) >> 5
            rows = jnp.full((16,), 0, jnp.int32) + j
            colbase = 64 * g2 + 4 * lanes
            plsc.store_scatter(idx_v, [rows, colbase], dow)
            plsc.store_scatter(idx_v, [rows, colbase + 1], month + 7)
            plsc.store_scatter(idx_v, [rows, colbase + 2], dom + 19)
            plsc.store_scatter(idx_v, [rows, colbase + 3], quarter + 50)
        pltpu.async_copy(
            table_hbm.at[idx_v.at[j]],
            dst_v.at[pl.ds(j * 128, 128)],
            sem,
        )
        return carry

    lax.fori_loop(0, _IDX_CHUNKS, chunk_step, 0)

    for _ in range(_IDX_CHUNKS):
        pltpu.make_async_copy(
            table_hbm.at[idx_v.at[0]],
            dst_v.at[pl.ds(0, 128)],
            sem,
        ).wait()

    pltpu.sync_copy(dst_v, out_hbm.at[pl.ds(wid * _IDX_PER_WORKER, _IDX_PER_WORKER)])


@functools.partial(jax.jit)
def _sc_lookup(ts, table):
    mesh = plsc.VectorSubcoreMesh(core_axis_name="c", subcore_axis_name="s")
    k = functools.partial(
        pl.kernel,
        mesh=mesh,
        out_type=jax.ShapeDtypeStruct((4 * _B, _SUB), jnp.float32),
        scratch_types=[
            pltpu.VMEM((_CHUNK,), jnp.int32),
            pltpu.VMEM((_IDX_CHUNKS, 128), jnp.int32),
            pltpu.VMEM((_IDX_PER_WORKER, _SUB), jnp.float32),
            pltpu.SemaphoreType.DMA,
        ],
        compiler_params=pltpu.CompilerParams(
            use_tc_tiling_on_sc=False, needs_layout_passes=False
        ),
    )(_body)
    return k(ts, table)


def kernel(timestamps, dow_table, month_table, dom_table, quarter_table):
    table = jnp.concatenate(
        [dow_table, month_table, dom_table, quarter_table], axis=0
    )
    ts = timestamps.astype(jnp.int32)
    out = _sc_lookup(ts, table)
    return out.reshape(_B, 4 * _SUB)


# speedup vs baseline: 2.6578x; 1.2018x over previous
import functools

import jax
import jax.numpy as jnp
from jax import lax
from jax.experimental import pallas as pl
from jax.experimental.pallas import tpu as pltpu
from jax.experimental.pallas import tpu_sc as plsc

_B = 16384
_SUB = 32
_TROWS = 54
_NUM_WORKERS = 32
_CHUNK = _B // _NUM_WORKERS
_GROUPS = _CHUNK // 16
_OUT_PER_WORKER = 4 * _CHUNK


def _body(ts_hbm, table_hbm, out_hbm, ts_v, table_v, dst_v, sem):
    wid = lax.axis_index("s") * 2 + lax.axis_index("c")
    base = wid * _CHUNK
    pltpu.sync_copy(table_hbm, table_v)
    pltpu.sync_copy(ts_hbm.at[pl.ds(base, _CHUNK)], ts_v)

    lanes = lax.iota(jnp.int32, 16)
    obase = wid * _OUT_PER_WORKER

    def group_step(g, carry):
        ts = ts_v[pl.ds(g * 16, 16)]
        dn0 = (ts.astype(jnp.float32) * jnp.float32(1.0 / 86400.0)).astype(
            jnp.int32
        )
        r = ts - dn0 * 86400
        dn = dn0 + jnp.where(r >= 86400, 1, 0) - jnp.where(r < 0, 1, 0)
        dow = dn - ((dn * 18725) >> 17) * 7
        doy = dn - ((dn * 22983) >> 23) * 365
        q30 = (doy * 1093) >> 15
        month = q30 - jnp.where(q30 >= 12, 12, 0)
        dom = doy - ((doy * 4229) >> 17) * 31
        quarter = (month * 11) >> 5

        rowbase = 64 * g + 4 * lanes
        for k, field in enumerate(
            (dow, month + 7, dom + 19, quarter + 50)
        ):
            orow = rowbase + k
            for c in range(_SUB):
                col = jnp.full((16,), c, jnp.int32)
                v = plsc.load_gather(table_v, [field, col])
                plsc.store_scatter(dst_v, [orow, col], v)
        pltpu.async_copy(
            dst_v.at[pl.ds(64 * g, 64)],
            out_hbm.at[pl.ds(obase + 64 * g, 64)],
            sem,
        )
        return carry

    lax.fori_loop(0, _GROUPS, group_step, 0)

    for _ in range(_GROUPS):
        pltpu.make_async_copy(
            dst_v.at[pl.ds(0, 64)],
            out_hbm.at[pl.ds(obase, 64)],
            sem,
        ).wait()


@functools.partial(jax.jit)
def _sc_lookup(ts, table):
    mesh = plsc.VectorSubcoreMesh(core_axis_name="c", subcore_axis_name="s")
    k = functools.partial(
        pl.kernel,
        mesh=mesh,
        out_type=jax.ShapeDtypeStruct((4 * _B, _SUB), jnp.float32),
        scratch_types=[
            pltpu.VMEM((_CHUNK,), jnp.int32),
            pltpu.VMEM((_TROWS, _SUB), jnp.float32),
            pltpu.VMEM((_OUT_PER_WORKER, _SUB), jnp.float32),
            pltpu.SemaphoreType.DMA,
        ],
        compiler_params=pltpu.CompilerParams(
            use_tc_tiling_on_sc=False, needs_layout_passes=False
        ),
    )(_body)
    return k(ts, table)


def kernel(timestamps, dow_table, month_table, dom_table, quarter_table):
    table = jnp.concatenate(
        [dow_table, month_table, dom_table, quarter_table], axis=0
    )
    ts = timestamps.astype(jnp.int32)
    out = _sc_lookup(ts, table)
    return out.reshape(_B, 4 * _SUB)


# speedup vs baseline: 3.5471x; 1.3346x over previous
import functools

import jax
import jax.numpy as jnp
from jax import lax
from jax.experimental import pallas as pl
from jax.experimental.pallas import tpu as pltpu
from jax.experimental.pallas import tpu_sc as plsc

_B = 16384
_SUB = 32
_TROWS = 54
_NUM_WORKERS = 32
_CHUNK = _B // _NUM_WORKERS
_GROUPS = _CHUNK // 16
_OUT_PER_WORKER = 4 * _CHUNK


def _body(ts_hbm, table_hbm, out_hbm, ts_v, table_v, dst_v, sem):
    wid = lax.axis_index("s") * 2 + lax.axis_index("c")
    base = wid * _CHUNK
    pltpu.sync_copy(table_hbm, table_v)
    pltpu.sync_copy(ts_hbm.at[pl.ds(base, _CHUNK)], ts_v)

    lanes = lax.iota(jnp.int32, 16)
    obase = wid * _OUT_PER_WORKER

    def group_step(g, carry):
        ts = ts_v[pl.ds(g * 16, 16)]
        dn0 = (ts.astype(jnp.float32) * jnp.float32(1.0 / 86400.0)).astype(
            jnp.int32
        )
        r = ts - dn0 * 86400
        dn = dn0 + jnp.where(r >= 86400, 1, 0) - jnp.where(r < 0, 1, 0)
        dow = dn - ((dn * 18725) >> 17) * 7
        doy = dn - ((dn * 22983) >> 23) * 365
        q30 = (doy * 1093) >> 15
        month = q30 - jnp.where(q30 >= 12, 12, 0)
        dom = doy - ((doy * 4229) >> 17) * 31
        quarter = (month * 11) >> 5

        rowbase = 64 * g + 4 * lanes
        for k, field in enumerate(
            (dow, month + 7, dom + 19, quarter + 50)
        ):
            orow = rowbase + k
            for cb in range(0, _SUB, 8):
                cols = [jnp.full((16,), c, jnp.int32) for c in range(cb, cb + 8)]
                vals = [plsc.load_gather(table_v, [field, col]) for col in cols]
                for col, v in zip(cols, vals):
                    plsc.store_scatter(dst_v, [orow, col], v)
        pltpu.async_copy(
            dst_v.at[pl.ds(64 * g, 64)],
            out_hbm.at[pl.ds(obase + 64 * g, 64)],
            sem,
        )
        return carry

    lax.fori_loop(0, _GROUPS, group_step, 0)

    for _ in range(_GROUPS):
        pltpu.make_async_copy(
            dst_v.at[pl.ds(0, 64)],
            out_hbm.at[pl.ds(obase, 64)],
            sem,
        ).wait()


@functools.partial(jax.jit)
def _sc_lookup(ts, table):
    mesh = plsc.VectorSubcoreMesh(core_axis_name="c", subcore_axis_name="s")
    k = functools.partial(
        pl.kernel,
        mesh=mesh,
        out_type=jax.ShapeDtypeStruct((4 * _B, _SUB), jnp.float32),
        scratch_types=[
            pltpu.VMEM((_CHUNK,), jnp.int32),
            pltpu.VMEM((_TROWS, _SUB), jnp.float32),
            pltpu.VMEM((_OUT_PER_WORKER, _SUB), jnp.float32),
            pltpu.SemaphoreType.DMA,
        ],
        compiler_params=pltpu.CompilerParams(
            use_tc_tiling_on_sc=False, needs_layout_passes=False
        ),
    )(_body)
    return k(ts, table)


def kernel(timestamps, dow_table, month_table, dom_table, quarter_table):
    table = jnp.concatenate(
        [dow_table, month_table, dom_table, quarter_table], axis=0
    )
    ts = timestamps.astype(jnp.int32)
    out = _sc_lookup(ts, table)
    return out.reshape(_B, 4 * _SUB)


# speedup vs baseline: 9.4504x; 2.6643x over previous
import functools

import jax
import jax.numpy as jnp
from jax import lax
from jax.experimental import pallas as pl
from jax.experimental.pallas import tpu as pltpu
from jax.experimental.pallas import tpu_sc as plsc

_B = 16384
_SUB = 32
_TROWS = 54
_NUM_WORKERS = 32
_CHUNK = _B // _NUM_WORKERS
_GROUPS = _CHUNK // 16
_OUT_PER_WORKER = 4 * _CHUNK * _SUB


def _body(ts_hbm, table_hbm, out_hbm, ts_v, table_v, dst_v, sem):
    wid = lax.axis_index("s") * 2 + lax.axis_index("c")
    base = wid * _CHUNK
    pltpu.sync_copy(table_hbm, table_v)
    pltpu.sync_copy(ts_hbm.at[pl.ds(base, _CHUNK)], ts_v)

    lanes = lax.iota(jnp.int32, 16)
    obase = wid * _OUT_PER_WORKER

    def group_step(g, carry):
        ts = ts_v[pl.ds(g * 16, 16)]
        dn0 = (ts.astype(jnp.float32) * jnp.float32(1.0 / 86400.0)).astype(
            jnp.int32
        )
        r = ts - dn0 * 86400
        dn = dn0 + jnp.where(r >= 86400, 1, 0) - jnp.where(r < 0, 1, 0)
        dow = dn - ((dn * 18725) >> 17) * 7
        doy = dn - ((dn * 22983) >> 23) * 365
        q30 = (doy * 1093) >> 15
        month = q30 - jnp.where(q30 >= 12, 12, 0)
        dom = doy - ((doy * 4229) >> 17) * 31
        quarter = (month * 11) >> 5

        addr = [
            dow * _SUB,
            (month + 7) * _SUB,
            (dom + 19) * _SUB,
            (quarter + 50) * _SUB,
        ]
        gbase = 64 * _SUB * g

        for b0 in range(0, 16, 2):
            vals = []
            for b in (b0, b0 + 1):
                for k in range(4):
                    rowoff = lax.gather(
                        addr[k],
                        jnp.full((16, 1), b, jnp.int32),
                        lax.GatherDimensionNumbers(
                            offset_dims=(),
                            collapsed_slice_dims=(0,),
                            start_index_map=(0,),
                        ),
                        (1,),
                        mode=lax.GatherScatterMode.PROMISE_IN_BOUNDS,
                    )
                    for m in (0, 16):
                        vals.append(
                            plsc.load_gather(table_v, [rowoff + (m + lanes)])
                        )
            i = 0
            for b in (b0, b0 + 1):
                dbase = gbase + 128 * b
                for k in range(4):
                    for m in (0, 16):
                        dst_v[pl.ds(dbase + 32 * k + m, 16)] = vals[i]
                        i += 1
        pltpu.async_copy(
            dst_v.at[pl.ds(gbase, 64 * _SUB)],
            out_hbm.at[pl.ds(obase + gbase, 64 * _SUB)],
            sem,
        )
        return carry

    lax.fori_loop(0, _GROUPS, group_step, 0)

    for _ in range(_GROUPS):
        pltpu.make_async_copy(
            dst_v.at[pl.ds(0, 64 * _SUB)],
            out_hbm.at[pl.ds(obase, 64 * _SUB)],
            sem,
        ).wait()


@functools.partial(jax.jit)
def _sc_lookup(ts, table_flat):
    mesh = plsc.VectorSubcoreMesh(core_axis_name="c", subcore_axis_name="s")
    k = functools.partial(
        pl.kernel,
        mesh=mesh,
        out_type=jax.ShapeDtypeStruct((4 * _B * _SUB,), jnp.float32),
        scratch_types=[
            pltpu.VMEM((_CHUNK,), jnp.int32),
            pltpu.VMEM((_TROWS * _SUB,), jnp.float32),
            pltpu.VMEM((_OUT_PER_WORKER,), jnp.float32),
            pltpu.SemaphoreType.DMA,
        ],
        compiler_params=pltpu.CompilerParams(
            use_tc_tiling_on_sc=False, needs_layout_passes=False
        ),
    )(_body)
    return k(ts, table_flat)


def kernel(timestamps, dow_table, month_table, dom_table, quarter_table):
    table = jnp.concatenate(
        [dow_table, month_table, dom_table, quarter_table], axis=0
    ).reshape(-1)
    ts = timestamps.astype(jnp.int32)
    out = _sc_lookup(ts, table)
    return out.reshape(_B, 4 * _SUB)


# speedup vs baseline: 9.5283x; 1.0082x over previous
import functools

import jax
import jax.numpy as jnp
from jax import lax
from jax.experimental import pallas as pl
from jax.experimental.pallas import tpu as pltpu
from jax.experimental.pallas import tpu_sc as plsc

_B = 16384
_SUB = 32
_TROWS = 54
_NUM_WORKERS = 32
_CHUNK = _B // _NUM_WORKERS
_GROUPS = _CHUNK // 16
_OUT_PER_WORKER = 4 * _CHUNK * _SUB


def _body(ts_hbm, table_hbm, out_hbm, ts_v, table_v, dst_v, sem):
    wid = lax.axis_index("s") * 2 + lax.axis_index("c")
    base = wid * _CHUNK
    pltpu.async_copy(table_hbm, table_v, sem)
    pltpu.async_copy(ts_hbm.at[pl.ds(base, _CHUNK)], ts_v, sem).wait()
    pltpu.make_async_copy(table_hbm, table_v, sem).wait()

    lanes = lax.iota(jnp.int32, 16)
    obase = wid * _OUT_PER_WORKER

    def group_step(g, carry):
        ts = ts_v[pl.ds(g * 16, 16)]
        dn0 = (ts.astype(jnp.float32) * jnp.float32(1.0 / 86400.0)).astype(
            jnp.int32
        )
        r = ts - dn0 * 86400
        dn = dn0 + jnp.where(r >= 86400, 1, 0) - jnp.where(r < 0, 1, 0)
        dow = dn - ((dn * 18725) >> 17) * 7
        doy = dn - ((dn * 22983) >> 23) * 365
        q30 = (doy * 1093) >> 15
        month = q30 - jnp.where(q30 >= 12, 12, 0)
        dom = doy - ((doy * 4229) >> 17) * 31
        quarter = (month * 11) >> 5

        addr = [
            dow * _SUB,
            (month + 7) * _SUB,
            (dom + 19) * _SUB,
            (quarter + 50) * _SUB,
        ]
        gbase = 64 * _SUB * g

        for b0 in range(0, 16, 2):
            vals = []
            for b in (b0, b0 + 1):
                for k in range(4):
                    rowoff = lax.gather(
                        addr[k],
                        jnp.full((16, 1), b, jnp.int32),
                        lax.GatherDimensionNumbers(
                            offset_dims=(),
                            collapsed_slice_dims=(0,),
                            start_index_map=(0,),
                        ),
                        (1,),
                        mode=lax.GatherScatterMode.PROMISE_IN_BOUNDS,
                    )
                    for m in (0, 16):
                        vals.append(
                            plsc.load_gather(table_v, [rowoff + (m + lanes)])
                        )
            i = 0
            for b in (b0, b0 + 1):
                dbase = gbase + 128 * b
                for k in range(4):
                    for m in (0, 16):
                        dst_v[pl.ds(dbase + 32 * k + m, 16)] = vals[i]
                        i += 1
        pltpu.async_copy(
            dst_v.at[pl.ds(gbase, 64 * _SUB)],
            out_hbm.at[pl.ds(obase + gbase, 64 * _SUB)],
            sem,
        )
        return carry

    lax.fori_loop(0, _GROUPS, group_step, 0)

    for _ in range(_GROUPS):
        pltpu.make_async_copy(
            dst_v.at[pl.ds(0, 64 * _SUB)],
            out_hbm.at[pl.ds(obase, 64 * _SUB)],
            sem,
        ).wait()


@functools.partial(jax.jit)
def _sc_lookup(ts, table_flat):
    mesh = plsc.VectorSubcoreMesh(core_axis_name="c", subcore_axis_name="s")
    k = functools.partial(
        pl.kernel,
        mesh=mesh,
        out_type=jax.ShapeDtypeStruct((4 * _B * _SUB,), jnp.float32),
        scratch_types=[
            pltpu.VMEM((_CHUNK,), jnp.int32),
            pltpu.VMEM((_TROWS * _SUB,), jnp.float32),
            pltpu.VMEM((_OUT_PER_WORKER,), jnp.float32),
            pltpu.SemaphoreType.DMA,
        ],
        compiler_params=pltpu.CompilerParams(
            use_tc_tiling_on_sc=False,
            needs_layout_passes=False,
        ),
    )(_body)
    return k(ts, table_flat)


def kernel(timestamps, dow_table, month_table, dom_table, quarter_table):
    table = jnp.concatenate(
        [dow_table, month_table, dom_table, quarter_table], axis=0
    ).reshape(-1)
    ts = timestamps.astype(jnp.int32)
    out = _sc_lookup(ts, table)
    return out.reshape(_B, 4 * _SUB)
